# Initial kernel scaffold; baseline (speedup 1.0000x reference)
import jax
import jax.numpy as jnp
from jax.experimental import pallas as pl


def kernel(state, slab):
    raise NotImplementedError("write your pallas kernel here")


# speedup vs baseline: 15.0970x; 15.0970x over previous
import functools

import numpy as np

import jax
import jax.numpy as jnp
from jax import lax
from jax.experimental import pallas as pl
from jax.experimental.pallas import tpu as pltpu

NEG_SLOPE = 0.2
BN_EPS = 1e-5
FEAT = 120
LANES = 128

W2_OFF = 48
WFC_OFF = 336
TAIL_OFF = 368
B1_ROW = 368
B2_ROW = 369
BFC_ROW = 370
GAMMA_ROW = 371
BETA_ROW = 372
SLAB_ROWS = 376


def _leaky(v):
    return jnp.where(v > 0, v, NEG_SLOPE * v)


def _w1_gather_index():
    idx = np.full((25, 128), 45, np.int32)
    for h in range(5):
        for w in range(5):
            p = h * 5 + w
            for ci in range(5):
                for ih in range(5):
                    for iw in range(5):
                        kh, kw = ih - h + 1, iw - w + 1
                        if 0 <= kh < 3 and 0 <= kw < 3:
                            idx[p, ci * 25 + ih * 5 + iw] = (kh * 3 + kw) * 5 + ci
    return idx.reshape(25 * 128)


_W1_IDX = _w1_gather_index()


def _encode(x, w1u_ref, slab_ref, tb):
    b1 = slab_ref[B1_ROW:B1_ROW + 1, 0:32]
    c1 = {}
    for h in range(5):
        for w in range(5):
            p = h * 5 + w
            c1[(h, w)] = _leaky(
                jnp.dot(x, w1u_ref[p * LANES:(p + 1) * LANES, :],
                        preferred_element_type=jnp.float32) + b1)

    pool1 = {(h, w): jnp.maximum(jnp.maximum(c1[(h, w)], c1[(h, w + 1)]),
                                 jnp.maximum(c1[(h + 1, w)], c1[(h + 1, w + 1)]))
             for h in range(4) for w in range(4)}

    acc = jnp.zeros((4 * tb, 32), jnp.float32)
    for kh in range(3):
        cols = []
        for kw in range(3):
            cols.append(jnp.concatenate(
                [pool1[(kh, kw)], pool1[(kh, kw + 1)],
                 pool1[(kh + 1, kw)], pool1[(kh + 1, kw + 1)]], axis=0))
        rows96 = jnp.concatenate(cols, axis=1)
        acc = acc + jnp.dot(
            rows96, slab_ref[W2_OFF + kh * 96:W2_OFF + (kh + 1) * 96, 0:32],
            preferred_element_type=jnp.float32)
    c2 = _leaky(acc + slab_ref[B2_ROW:B2_ROW + 1, 0:32])

    feat = jnp.maximum(jnp.maximum(c2[0:tb], c2[tb:2 * tb]),
                       jnp.maximum(c2[2 * tb:3 * tb], c2[3 * tb:4 * tb]))

    return (jnp.dot(feat, slab_ref[WFC_OFF:WFC_OFF + 32, :],
                    preferred_element_type=jnp.float32)
            + slab_ref[BFC_ROW:BFC_ROW + 1, :])


def _pass1(state_ref, w1u_ref, slab_ref, y_ref, stats_ref, *, tb, n_valid):
    x = state_ref[...]
    x = jnp.pad(x, ((0, 0), (0, LANES - x.shape[1])))
    y = _encode(x, w1u_ref, slab_ref, tb)
    y_ref[...] = y
    if n_valid is None:
        yv = y
    else:
        gid = pl.program_id(0) * tb + lax.broadcasted_iota(jnp.int32, y.shape, 0)
        yv = jnp.where(gid < n_valid, y, 0.0)
    s = jnp.sum(yv, axis=0, keepdims=True)
    ss = jnp.sum(yv * y, axis=0, keepdims=True)
    rid = lax.broadcasted_iota(jnp.int32, stats_ref.shape, 0)
    stats_ref[...] = jnp.where(rid == 0, s, 0.0) + jnp.where(rid == 1, ss, 0.0)


def _pass2(y_ref, tot_ref, tail_ref, out_ref, *, inv_n):
    mean = tot_ref[0:1, :] * inv_n
    var = jnp.maximum(tot_ref[1:2, :] * inv_n - mean * mean, 0.0)
    y_hat = (y_ref[...] - mean) * lax.rsqrt(var + BN_EPS)
    o = _leaky(y_hat * tail_ref[GAMMA_ROW - TAIL_OFF:GAMMA_ROW - TAIL_OFF + 1, :]
               + tail_ref[BETA_ROW - TAIL_OFF:BETA_ROW - TAIL_OFF + 1, :])
    out_ref[...] = o[:, 0:FEAT]


_MOSAIC = dict(vmem_limit_bytes=48 * 1024 * 1024)


@jax.jit
def _forward(state, slab):
    B = state.shape[0]
    b_pad = ((B + 7) // 8) * 8
    tb = next(t for t in (512, 256, 128, 64, 32, 16, 8) if b_pad % t == 0)
    nb = b_pad // tb
    if b_pad != B:
        state = jnp.pad(state, ((0, b_pad - B), (0, 0)))
    n_valid = None if b_pad == B else B

    w1u = jnp.take(slab[0:48, 0:32], _W1_IDX, axis=0)

    flops = 2 * b_pad * (25 * LANES * 32 + 4 * 288 * 32 + 32 * LANES)
    bytes1 = 4 * (b_pad * 125 + 25 * LANES * 32 + SLAB_ROWS * LANES
                  + b_pad * LANES + nb * 8 * LANES)
    y, stats = pl.pallas_call(
        functools.partial(_pass1, tb=tb, n_valid=n_valid),
        grid=(nb,),
        in_specs=[pl.BlockSpec((tb, 125), lambda i: (i, 0)),
                  pl.BlockSpec((25 * LANES, 32), lambda i: (0, 0)),
                  pl.BlockSpec((SLAB_ROWS, LANES), lambda i: (0, 0))],
        out_specs=[pl.BlockSpec((tb, LANES), lambda i: (i, 0)),
                   pl.BlockSpec((8, LANES), lambda i: (i, 0))],
        out_shape=(jax.ShapeDtypeStruct((b_pad, LANES), jnp.float32),
                   jax.ShapeDtypeStruct((nb * 8, LANES), jnp.float32)),
        compiler_params=pltpu.CompilerParams(
            dimension_semantics=("parallel",), **_MOSAIC),
        cost_estimate=pl.CostEstimate(flops=flops, transcendentals=0,
                                      bytes_accessed=bytes1),
    )(state, w1u, slab)

    totals = jnp.sum(stats.reshape(nb, 8, LANES), axis=0)

    out = pl.pallas_call(
        functools.partial(_pass2, inv_n=1.0 / B),
        grid=(nb,),
        in_specs=[pl.BlockSpec((tb, LANES), lambda i: (i, 0)),
                  pl.BlockSpec((8, LANES), lambda i: (0, 0)),
                  pl.BlockSpec((8, LANES), lambda i: (TAIL_OFF // 8, 0))],
        out_specs=pl.BlockSpec((tb, FEAT), lambda i: (i, 0)),
        out_shape=jax.ShapeDtypeStruct((b_pad, FEAT), jnp.float32),
        compiler_params=pltpu.CompilerParams(
            dimension_semantics=("parallel",), **_MOSAIC),
        cost_estimate=pl.CostEstimate(
            flops=10 * b_pad * LANES, transcendentals=LANES,
            bytes_accessed=4 * (b_pad * LANES + b_pad * FEAT)),
    )(y, totals, slab)

    return out[:B] if b_pad != B else out


def kernel(state, slab):
    return _forward(state, slab)


# speedup vs baseline: 16.7809x; 1.1115x over previous
import functools

import numpy as np

import jax
import jax.numpy as jnp
from jax import lax
from jax.experimental import pallas as pl
from jax.experimental.pallas import tpu as pltpu

NEG_SLOPE = 0.2
BN_EPS = 1e-5
FEAT = 120
LANES = 128

W2_OFF = 48
WFC_OFF = 336
TAIL_OFF = 368
B1_ROW = 368
B2_ROW = 369
BFC_ROW = 370
GAMMA_ROW = 371
BETA_ROW = 372
SLAB_ROWS = 376


def _leaky(v):
    return jnp.where(v > 0, v, NEG_SLOPE * v)


def _w1_gather_index():
    idx = np.full((25, 128), 45, np.int32)
    for h in range(5):
        for w in range(5):
            p = h * 5 + w
            for ci in range(5):
                for ih in range(5):
                    for iw in range(5):
                        kh, kw = ih - h + 1, iw - w + 1
                        if 0 <= kh < 3 and 0 <= kw < 3:
                            idx[p, ci * 25 + ih * 5 + iw] = (kh * 3 + kw) * 5 + ci
    return idx.reshape(25 * 128)


_W1_IDX = _w1_gather_index()


def _encode(x, w1u_ref, slab_ref, tb):
    b1 = slab_ref[B1_ROW:B1_ROW + 1, 0:32]
    c1 = {}
    for h in range(5):
        for w in range(5):
            p = h * 5 + w
            c1[(h, w)] = _leaky(
                jnp.dot(x, w1u_ref[p * LANES:(p + 1) * LANES, :],
                        preferred_element_type=jnp.float32) + b1)

    pool1 = {(h, w): jnp.maximum(jnp.maximum(c1[(h, w)], c1[(h, w + 1)]),
                                 jnp.maximum(c1[(h + 1, w)], c1[(h + 1, w + 1)]))
             for h in range(4) for w in range(4)}

    acc = jnp.zeros((4 * tb, 32), jnp.float32)
    for kh in range(3):
        cols = []
        for kw in range(3):
            cols.append(jnp.concatenate(
                [pool1[(kh, kw)], pool1[(kh, kw + 1)],
                 pool1[(kh + 1, kw)], pool1[(kh + 1, kw + 1)]], axis=0))
        rows96 = jnp.concatenate(cols, axis=1)
        acc = acc + jnp.dot(
            rows96, slab_ref[W2_OFF + kh * 96:W2_OFF + (kh + 1) * 96, 0:32],
            preferred_element_type=jnp.float32)
    c2 = _leaky(acc + slab_ref[B2_ROW:B2_ROW + 1, 0:32])

    feat = jnp.maximum(jnp.maximum(c2[0:tb], c2[tb:2 * tb]),
                       jnp.maximum(c2[2 * tb:3 * tb], c2[3 * tb:4 * tb]))

    return (jnp.dot(feat, slab_ref[WFC_OFF:WFC_OFF + 32, :],
                    preferred_element_type=jnp.float32)
            + slab_ref[BFC_ROW:BFC_ROW + 1, :])


def _pass1(state_ref, w1u_ref, slab_ref, y_ref, stats_ref, *, tb, n_valid):
    x = state_ref[...]
    x = jnp.pad(x, ((0, 0), (0, LANES - x.shape[1])))
    y = _encode(x, w1u_ref, slab_ref, tb)
    y_ref[...] = y
    if n_valid is None:
        yv = y
    else:
        gid = pl.program_id(0) * tb + lax.broadcasted_iota(jnp.int32, y.shape, 0)
        yv = jnp.where(gid < n_valid, y, 0.0)
    s = jnp.sum(yv, axis=0, keepdims=True)
    ss = jnp.sum(yv * y, axis=0, keepdims=True)
    rid = lax.broadcasted_iota(jnp.int32, stats_ref.shape, 0)
    stats_ref[...] = jnp.where(rid == 0, s, 0.0) + jnp.where(rid == 1, ss, 0.0)


def _pass2(y_ref, tot_ref, tail_ref, out_ref, *, inv_n):
    mean = tot_ref[0:1, :] * inv_n
    var = jnp.maximum(tot_ref[1:2, :] * inv_n - mean * mean, 0.0)
    y_hat = (y_ref[...] - mean) * lax.rsqrt(var + BN_EPS)
    o = _leaky(y_hat * tail_ref[GAMMA_ROW - TAIL_OFF:GAMMA_ROW - TAIL_OFF + 1, :]
               + tail_ref[BETA_ROW - TAIL_OFF:BETA_ROW - TAIL_OFF + 1, :])
    out_ref[...] = o[:, 0:FEAT]


_MOSAIC = dict(vmem_limit_bytes=48 * 1024 * 1024)


@jax.jit
def _forward(state, slab):
    B = state.shape[0]
    b_pad = ((B + 7) // 8) * 8
    tb = next(t for t in (1024, 512, 256, 128, 64, 32, 16, 8) if b_pad % t == 0)
    nb = b_pad // tb
    if b_pad != B:
        state = jnp.pad(state, ((0, b_pad - B), (0, 0)))
    n_valid = None if b_pad == B else B

    w1u = jnp.take(slab[0:48, 0:32], _W1_IDX, axis=0)

    flops = 2 * b_pad * (25 * LANES * 32 + 4 * 288 * 32 + 32 * LANES)
    bytes1 = 4 * (b_pad * 125 + 25 * LANES * 32 + SLAB_ROWS * LANES
                  + b_pad * LANES + nb * 8 * LANES)
    y, stats = pl.pallas_call(
        functools.partial(_pass1, tb=tb, n_valid=n_valid),
        grid=(nb,),
        in_specs=[pl.BlockSpec((tb, 125), lambda i: (i, 0)),
                  pl.BlockSpec((25 * LANES, 32), lambda i: (0, 0)),
                  pl.BlockSpec((SLAB_ROWS, LANES), lambda i: (0, 0))],
        out_specs=[pl.BlockSpec((tb, LANES), lambda i: (i, 0)),
                   pl.BlockSpec((8, LANES), lambda i: (i, 0))],
        out_shape=(jax.ShapeDtypeStruct((b_pad, LANES), jnp.float32),
                   jax.ShapeDtypeStruct((nb * 8, LANES), jnp.float32)),
        compiler_params=pltpu.CompilerParams(
            dimension_semantics=("parallel",), **_MOSAIC),
        cost_estimate=pl.CostEstimate(flops=flops, transcendentals=0,
                                      bytes_accessed=bytes1),
    )(state, w1u, slab)

    totals = jnp.sum(stats.reshape(nb, 8, LANES), axis=0)

    out = pl.pallas_call(
        functools.partial(_pass2, inv_n=1.0 / B),
        grid=(nb,),
        in_specs=[pl.BlockSpec((tb, LANES), lambda i: (i, 0)),
                  pl.BlockSpec((8, LANES), lambda i: (0, 0)),
                  pl.BlockSpec((8, LANES), lambda i: (TAIL_OFF // 8, 0))],
        out_specs=pl.BlockSpec((tb, FEAT), lambda i: (i, 0)),
        out_shape=jax.ShapeDtypeStruct((b_pad, FEAT), jnp.float32),
        compiler_params=pltpu.CompilerParams(
            dimension_semantics=("parallel",), **_MOSAIC),
        cost_estimate=pl.CostEstimate(
            flops=10 * b_pad * LANES, transcendentals=LANES,
            bytes_accessed=4 * (b_pad * LANES + b_pad * FEAT)),
    )(y, totals, slab)

    return out[:B] if b_pad != B else out


def kernel(state, slab):
    return _forward(state, slab)


# speedup vs baseline: 36.6636x; 2.1848x over previous
import functools

import numpy as np

import jax
import jax.numpy as jnp
from jax import lax
from jax.experimental import pallas as pl
from jax.experimental.pallas import tpu as pltpu

NEG_SLOPE = 0.2
BN_EPS = 1e-5
FEAT = 120
LANES = 128

W2_OFF = 48
WFC_OFF = 336
B1_ROW = 368
B2_ROW = 369
BFC_ROW = 370
GAMMA_ROW = 371
BETA_ROW = 372

W1T_OFF = 0
W2T_OFF = 800
WFCT_OFF = 896


def _leaky(v):
    return jnp.where(v > 0, v, NEG_SLOPE * v)


def _w1_gather_index():
    idx = np.full((25, 128), 45, np.int32)
    for h in range(5):
        for w in range(5):
            p = h * 5 + w
            for ci in range(5):
                for ih in range(5):
                    for iw in range(5):
                        kh, kw = ih - h + 1, iw - w + 1
                        if 0 <= kh < 3 and 0 <= kw < 3:
                            idx[p, ci * 25 + ih * 5 + iw] = (kh * 3 + kw) * 5 + ci
    return idx.reshape(25 * 128)


_W1_IDX = _w1_gather_index()


def _pass1(state_ref, w_ref, y_ref, stats_ref, pool_ref, *, tb, n_valid):
    x = state_ref[...]
    x = jnp.pad(x, ((0, 0), (0, LANES - x.shape[1])))
    lane = lax.broadcasted_iota(jnp.int32, x.shape, 1)
    x = jnp.where(lane == 125, 1.0, x)
    xt = jnp.transpose(x)

    c1 = jnp.dot(w_ref[W1T_OFF:W1T_OFF + 800, :], xt,
                 preferred_element_type=jnp.float32)

    for ph in range(4):
        for pw in range(4):
            nw, ne = (ph * 5 + pw) * 32, (ph * 5 + pw + 1) * 32
            sw, se = ((ph + 1) * 5 + pw) * 32, ((ph + 1) * 5 + pw + 1) * 32
            t = jnp.maximum(jnp.maximum(c1[nw:nw + 32], c1[ne:ne + 32]),
                            jnp.maximum(c1[sw:sw + 32], c1[se:se + 32]))
            pool_ref[(ph * 4 + pw) * 32:(ph * 4 + pw + 1) * 32, :] = _leaky(t)

    c2 = {}
    for oh in range(2):
        for ow in range(2):
            acc = jnp.zeros((32, tb), jnp.float32)
            for kh in range(3):
                r0 = (oh + kh) * 128 + ow * 32
                acc = acc + jnp.dot(
                    w_ref[W2T_OFF + kh * 32:W2T_OFF + (kh + 1) * 32, 0:96],
                    pool_ref[r0:r0 + 96, :],
                    preferred_element_type=jnp.float32)
            c2[(oh, ow)] = acc

    feat = jnp.maximum(jnp.maximum(c2[(0, 0)], c2[(0, 1)]),
                       jnp.maximum(c2[(1, 0)], c2[(1, 1)]))
    feat = _leaky(feat + w_ref[W2T_OFF:W2T_OFF + 32, 96:97])

    rid = lax.broadcasted_iota(jnp.int32, feat.shape, 0)
    feat = jnp.where(rid == 30, 1.0, feat)
    y = jnp.dot(w_ref[WFCT_OFF:WFCT_OFF + 128, 0:32], feat,
                preferred_element_type=jnp.float32)
    y_ref[...] = y

    if n_valid is None:
        yv = y
    else:
        gid = pl.program_id(0) * tb + lax.broadcasted_iota(jnp.int32, y.shape, 1)
        yv = jnp.where(gid < n_valid, y, 0.0)
    s = jnp.sum(yv, axis=1, keepdims=True)
    ss = jnp.sum(yv * y, axis=1, keepdims=True)
    lid = lax.broadcasted_iota(jnp.int32, stats_ref.shape, 1)
    stats_ref[...] = jnp.where(lid == 0, s, 0.0) + jnp.where(lid == 1, ss, 0.0)


def _pass2(y_ref, aux_ref, out_ref, *, inv_n):
    mean = aux_ref[:, 0:1] * inv_n
    var = jnp.maximum(aux_ref[:, 1:2] * inv_n - mean * mean, 0.0)
    y_hat = (y_ref[...] - mean) * lax.rsqrt(var + BN_EPS)
    o = _leaky(y_hat * aux_ref[:, 2:3] + aux_ref[:, 3:4])
    out_ref[...] = jnp.transpose(o)[:, 0:FEAT]


_MOSAIC = dict(vmem_limit_bytes=48 * 1024 * 1024)


@jax.jit
def _forward(state, slab):
    B = state.shape[0]
    b_pad = ((B + 7) // 8) * 8
    tb = next(t for t in (1024, 512, 256, 128) if b_pad % t == 0) \
        if b_pad % 128 == 0 else b_pad
    nb = b_pad // tb
    if b_pad != B:
        state = jnp.pad(state, ((0, b_pad - B), (0, 0)))
    n_valid = None if b_pad == B else B
    lane = jnp.arange(LANES)[None, :]

    w1u = jnp.take(slab[0:48, 0:32], _W1_IDX, axis=0)
    w1t = w1u.reshape(25, 128, 32).transpose(0, 2, 1).reshape(800, 128)
    w1t = jnp.where(lane == 125, jnp.tile(slab[B1_ROW, 0:32], 25)[:, None], w1t)
    w2t = slab[W2_OFF:WFC_OFF, 0:32].reshape(3, 96, 32)
    w2t = w2t.transpose(0, 2, 1).reshape(96, 96)
    w2t = jnp.pad(w2t, ((0, 0), (0, 32)))
    w2t = jnp.where(lane == 96, jnp.tile(slab[B2_ROW, 0:32], 3)[:, None], w2t)
    wfct = jnp.transpose(slab[WFC_OFF:WFC_OFF + 32, :])
    wfct = jnp.where(jnp.arange(32)[None, :] == 30, slab[BFC_ROW, :][:, None],
                     wfct)
    wfct = jnp.pad(wfct, ((0, 0), (0, 96)))
    wts = jnp.concatenate([w1t, w2t, wfct], axis=0)

    flops = 2 * b_pad * (800 * LANES + 12 * 32 * 96 + LANES * 32)
    bytes1 = 4 * (b_pad * 125 + 1024 * LANES + b_pad * LANES + nb * LANES * LANES)
    y, stats = pl.pallas_call(
        functools.partial(_pass1, tb=tb, n_valid=n_valid),
        grid=(nb,),
        in_specs=[pl.BlockSpec((tb, 125), lambda i: (i, 0)),
                  pl.BlockSpec((1024, LANES), lambda i: (0, 0))],
        out_specs=[pl.BlockSpec((LANES, tb), lambda i: (0, i)),
                   pl.BlockSpec((LANES, LANES), lambda i: (0, i))],
        out_shape=(jax.ShapeDtypeStruct((LANES, b_pad), jnp.float32),
                   jax.ShapeDtypeStruct((LANES, nb * LANES), jnp.float32)),
        scratch_shapes=[pltpu.VMEM((512, tb), jnp.float32)],
        compiler_params=pltpu.CompilerParams(
            dimension_semantics=("parallel",), **_MOSAIC),
        cost_estimate=pl.CostEstimate(flops=flops, transcendentals=0,
                                      bytes_accessed=bytes1),
    )(state, wts)

    totals = jnp.sum(stats.reshape(LANES, nb, LANES), axis=1)
    aux = jnp.concatenate(
        [totals[:, 0:1], totals[:, 1:2],
         slab[GAMMA_ROW, :][:, None], slab[BETA_ROW, :][:, None]], axis=1)
    aux = jnp.pad(aux, ((0, 0), (0, LANES - 4)))

    out = pl.pallas_call(
        functools.partial(_pass2, inv_n=1.0 / B),
        grid=(nb,),
        in_specs=[pl.BlockSpec((LANES, tb), lambda i: (0, i)),
                  pl.BlockSpec((LANES, LANES), lambda i: (0, 0))],
        out_specs=pl.BlockSpec((tb, FEAT), lambda i: (i, 0)),
        out_shape=jax.ShapeDtypeStruct((b_pad, FEAT), jnp.float32),
        compiler_params=pltpu.CompilerParams(
            dimension_semantics=("parallel",), **_MOSAIC),
        cost_estimate=pl.CostEstimate(
            flops=10 * b_pad * LANES, transcendentals=LANES,
            bytes_accessed=4 * (b_pad * LANES + b_pad * FEAT)),
    )(y, aux)

    return out[:B] if b_pad != B else out


def kernel(state, slab):
    return _forward(state, slab)


# speedup vs baseline: 47.2527x; 1.2888x over previous
import functools

import numpy as np

import jax
import jax.numpy as jnp
from jax import lax
from jax.experimental import pallas as pl
from jax.experimental.pallas import tpu as pltpu

NEG_SLOPE = 0.2
BN_EPS = 1e-5
FEAT = 120
LANES = 128

W2_OFF = 48
WFC_OFF = 336
B1_ROW = 368
B2_ROW = 369
BFC_ROW = 370
GAMMA_ROW = 371
BETA_ROW = 372

W1T_OFF = 0
W2T_OFF = 800
WFCT_OFF = 896


def _leaky(v):
    return jnp.where(v > 0, v, NEG_SLOPE * v)


def _w1_gather_index():
    idx = np.full((25, 128), 45, np.int32)
    for h in range(5):
        for w in range(5):
            p = h * 5 + w
            for ci in range(5):
                for ih in range(5):
                    for iw in range(5):
                        kh, kw = ih - h + 1, iw - w + 1
                        if 0 <= kh < 3 and 0 <= kw < 3:
                            idx[p, ci * 25 + ih * 5 + iw] = (kh * 3 + kw) * 5 + ci
    return idx.reshape(25 * 128)


_W1_IDX = _w1_gather_index()


def _pass1(state_ref, w_ref, y_ref, stats_ref, pool_ref, *, tb, n_valid):
    x = state_ref[...]
    x = jnp.pad(x, ((0, 0), (0, LANES - x.shape[1])))
    lane = lax.broadcasted_iota(jnp.int32, x.shape, 1)
    x = jnp.where(lane == 125, 1.0, x)
    xt = jnp.transpose(x)

    c1 = jnp.dot(w_ref[W1T_OFF:W1T_OFF + 800, :], xt,
                 preferred_element_type=jnp.float32)

    for ph in range(4):
        for pw in range(4):
            nw, ne = (ph * 5 + pw) * 32, (ph * 5 + pw + 1) * 32
            sw, se = ((ph + 1) * 5 + pw) * 32, ((ph + 1) * 5 + pw + 1) * 32
            t = jnp.maximum(jnp.maximum(c1[nw:nw + 32], c1[ne:ne + 32]),
                            jnp.maximum(c1[sw:sw + 32], c1[se:se + 32]))
            pool_ref[(ph * 4 + pw) * 32:(ph * 4 + pw + 1) * 32, :] = _leaky(t)

    c2 = {}
    for oh in range(2):
        for ow in range(2):
            acc = jnp.zeros((32, tb), jnp.float32)
            for kh in range(3):
                r0 = (oh + kh) * 128 + ow * 32
                acc = acc + jnp.dot(
                    w_ref[W2T_OFF + kh * 32:W2T_OFF + (kh + 1) * 32, 0:96],
                    pool_ref[r0:r0 + 96, :],
                    preferred_element_type=jnp.float32)
            c2[(oh, ow)] = acc

    feat = jnp.maximum(jnp.maximum(c2[(0, 0)], c2[(0, 1)]),
                       jnp.maximum(c2[(1, 0)], c2[(1, 1)]))
    feat = _leaky(feat + w_ref[W2T_OFF:W2T_OFF + 32, 96:97])

    rid = lax.broadcasted_iota(jnp.int32, feat.shape, 0)
    feat = jnp.where(rid == 30, 1.0, feat)
    y = jnp.dot(w_ref[WFCT_OFF:WFCT_OFF + 128, 0:32], feat,
                preferred_element_type=jnp.float32)
    y_ref[...] = y

    if n_valid is None:
        yv = y
    else:
        gid = pl.program_id(0) * tb + lax.broadcasted_iota(jnp.int32, y.shape, 1)
        yv = jnp.where(gid < n_valid, y, 0.0)
    s = jnp.sum(yv, axis=1, keepdims=True)
    ss = jnp.sum(yv * y, axis=1, keepdims=True)
    lid = lax.broadcasted_iota(jnp.int32, stats_ref.shape, 1)
    stats_ref[...] = jnp.where(lid == 0, s, 0.0) + jnp.where(lid == 1, ss, 0.0)


def _pass2(y_ref, aux_ref, out_ref, *, inv_n):
    mean = aux_ref[:, 0:1] * inv_n
    var = jnp.maximum(aux_ref[:, 1:2] * inv_n - mean * mean, 0.0)
    y_hat = (y_ref[...] - mean) * lax.rsqrt(var + BN_EPS)
    o = _leaky(y_hat * aux_ref[:, 2:3] + aux_ref[:, 3:4])
    out_ref[...] = jnp.transpose(o)[:, 0:FEAT]


_MOSAIC = dict(vmem_limit_bytes=48 * 1024 * 1024)


@jax.jit
def _forward(state, slab):
    B = state.shape[0]
    b_pad = ((B + 7) // 8) * 8
    tb = next(t for t in (2048, 1024, 512, 256, 128) if b_pad % t == 0) \
        if b_pad % 128 == 0 else b_pad
    nb = b_pad // tb
    tb2 = next(t for t in (4096, 2048, 1024, 512, 256, 128) if b_pad % t == 0) \
        if b_pad % 128 == 0 else b_pad
    nb2 = b_pad // tb2
    if b_pad != B:
        state = jnp.pad(state, ((0, b_pad - B), (0, 0)))
    n_valid = None if b_pad == B else B
    lane = jnp.arange(LANES)[None, :]

    w1u = jnp.take(slab[0:48, 0:32], _W1_IDX, axis=0)
    w1t = w1u.reshape(25, 128, 32).transpose(0, 2, 1).reshape(800, 128)
    w1t = jnp.where(lane == 125, jnp.tile(slab[B1_ROW, 0:32], 25)[:, None], w1t)
    w2t = slab[W2_OFF:WFC_OFF, 0:32].reshape(3, 96, 32)
    w2t = w2t.transpose(0, 2, 1).reshape(96, 96)
    w2t = jnp.pad(w2t, ((0, 0), (0, 32)))
    w2t = jnp.where(lane == 96, jnp.tile(slab[B2_ROW, 0:32], 3)[:, None], w2t)
    wfct = jnp.transpose(slab[WFC_OFF:WFC_OFF + 32, :])
    wfct = jnp.where(jnp.arange(32)[None, :] == 30, slab[BFC_ROW, :][:, None],
                     wfct)
    wfct = jnp.pad(wfct, ((0, 0), (0, 96)))
    wts = jnp.concatenate([w1t, w2t, wfct], axis=0)

    flops = 2 * b_pad * (800 * LANES + 12 * 32 * 96 + LANES * 32)
    bytes1 = 4 * (b_pad * 125 + 1024 * LANES + b_pad * LANES + nb * LANES * LANES)
    y, stats = pl.pallas_call(
        functools.partial(_pass1, tb=tb, n_valid=n_valid),
        grid=(nb,),
        in_specs=[pl.BlockSpec((tb, 125), lambda i: (i, 0)),
                  pl.BlockSpec((1024, LANES), lambda i: (0, 0))],
        out_specs=[pl.BlockSpec((LANES, tb), lambda i: (0, i)),
                   pl.BlockSpec((LANES, LANES), lambda i: (0, i))],
        out_shape=(jax.ShapeDtypeStruct((LANES, b_pad), jnp.float32),
                   jax.ShapeDtypeStruct((LANES, nb * LANES), jnp.float32)),
        scratch_shapes=[pltpu.VMEM((512, tb), jnp.float32)],
        compiler_params=pltpu.CompilerParams(
            dimension_semantics=("parallel",), **_MOSAIC),
        cost_estimate=pl.CostEstimate(flops=flops, transcendentals=0,
                                      bytes_accessed=bytes1),
    )(state, wts)

    totals = jnp.sum(stats.reshape(LANES, nb, LANES), axis=1)
    aux = jnp.concatenate(
        [totals[:, 0:1], totals[:, 1:2],
         slab[GAMMA_ROW, :][:, None], slab[BETA_ROW, :][:, None]], axis=1)
    aux = jnp.pad(aux, ((0, 0), (0, LANES - 4)))

    out = pl.pallas_call(
        functools.partial(_pass2, inv_n=1.0 / B),
        grid=(nb2,),
        in_specs=[pl.BlockSpec((LANES, tb2), lambda i: (0, i)),
                  pl.BlockSpec((LANES, LANES), lambda i: (0, 0))],
        out_specs=pl.BlockSpec((tb2, FEAT), lambda i: (i, 0)),
        out_shape=jax.ShapeDtypeStruct((b_pad, FEAT), jnp.float32),
        compiler_params=pltpu.CompilerParams(
            dimension_semantics=("parallel",), **_MOSAIC),
        cost_estimate=pl.CostEstimate(
            flops=10 * b_pad * LANES, transcendentals=LANES,
            bytes_accessed=4 * (b_pad * LANES + b_pad * FEAT)),
    )(y, aux)

    return out[:B] if b_pad != B else out


def kernel(state, slab):
    return _forward(state, slab)


# speedup vs baseline: 50.8803x; 1.0768x over previous
import functools

import numpy as np

import jax
import jax.numpy as jnp
from jax import lax
from jax.experimental import pallas as pl
from jax.experimental.pallas import tpu as pltpu

NEG_SLOPE = 0.2
BN_EPS = 1e-5
FEAT = 120
LANES = 128

W2_OFF = 48
WFC_OFF = 336
B1_ROW = 368
B2_ROW = 369
BFC_ROW = 370
GAMMA_ROW = 371
BETA_ROW = 372

W1T_OFF = 0
W2T_OFF = 800
WFCT_OFF = 896


def _leaky(v):
    return jnp.where(v > 0, v, NEG_SLOPE * v)


def _w1_gather_index():
    idx = np.full((25, 128), 45, np.int32)
    for h in range(5):
        for w in range(5):
            p = h * 5 + w
            for ci in range(5):
                for ih in range(5):
                    for iw in range(5):
                        kh, kw = ih - h + 1, iw - w + 1
                        if 0 <= kh < 3 and 0 <= kw < 3:
                            idx[p, ci * 25 + ih * 5 + iw] = (kh * 3 + kw) * 5 + ci
    return idx.reshape(25 * 128)


_W1_IDX = _w1_gather_index()


def _pass1(state_ref, w_ref, y_ref, stats_ref, pool_ref, *, tb, n_valid):
    x = state_ref[...]
    x = jnp.pad(x, ((0, 0), (0, LANES - x.shape[1])))
    lane = lax.broadcasted_iota(jnp.int32, x.shape, 1)
    x = jnp.where(lane == 125, 1.0, x)
    xt = jnp.transpose(x)

    c1 = jnp.dot(w_ref[W1T_OFF:W1T_OFF + 800, :], xt,
                 preferred_element_type=jnp.float32)

    for ph in range(4):
        for pw in range(4):
            nw, ne = (ph * 5 + pw) * 32, (ph * 5 + pw + 1) * 32
            sw, se = ((ph + 1) * 5 + pw) * 32, ((ph + 1) * 5 + pw + 1) * 32
            t = jnp.maximum(jnp.maximum(c1[nw:nw + 32], c1[ne:ne + 32]),
                            jnp.maximum(c1[sw:sw + 32], c1[se:se + 32]))
            pool_ref[(ph * 4 + pw) * 32:(ph * 4 + pw + 1) * 32, :] = _leaky(t)

    c2 = {}
    for oh in range(2):
        for ow in range(2):
            acc = jnp.zeros((32, tb), jnp.float32)
            for kh in range(3):
                r0 = (oh + kh) * 128 + ow * 32
                acc = acc + jnp.dot(
                    w_ref[W2T_OFF + kh * 32:W2T_OFF + (kh + 1) * 32, 0:96],
                    pool_ref[r0:r0 + 96, :],
                    preferred_element_type=jnp.float32)
            c2[(oh, ow)] = acc

    feat = jnp.maximum(jnp.maximum(c2[(0, 0)], c2[(0, 1)]),
                       jnp.maximum(c2[(1, 0)], c2[(1, 1)]))
    feat = _leaky(feat + w_ref[W2T_OFF:W2T_OFF + 32, 96:97])

    rid = lax.broadcasted_iota(jnp.int32, feat.shape, 0)
    feat = jnp.where(rid == 30, 1.0, feat)
    y = jnp.dot(w_ref[WFCT_OFF:WFCT_OFF + 128, 0:32], feat,
                preferred_element_type=jnp.float32)
    y_ref[...] = y.astype(jnp.bfloat16)

    if n_valid is None:
        yv = y
    else:
        gid = pl.program_id(0) * tb + lax.broadcasted_iota(jnp.int32, y.shape, 1)
        yv = jnp.where(gid < n_valid, y, 0.0)
    s = jnp.sum(yv, axis=1, keepdims=True)
    ss = jnp.sum(yv * y, axis=1, keepdims=True)
    lid = lax.broadcasted_iota(jnp.int32, stats_ref.shape, 1)
    stats_ref[...] = jnp.where(lid == 0, s, 0.0) + jnp.where(lid == 1, ss, 0.0)


def _pass2(y_ref, aux_ref, out_ref, *, inv_n):
    mean = aux_ref[:, 0:1] * inv_n
    var = jnp.maximum(aux_ref[:, 1:2] * inv_n - mean * mean, 0.0)
    y_hat = (y_ref[...].astype(jnp.float32) - mean) * lax.rsqrt(var + BN_EPS)
    o = _leaky(y_hat * aux_ref[:, 2:3] + aux_ref[:, 3:4])
    out_ref[...] = jnp.transpose(o)[:, 0:FEAT]


_MOSAIC = dict(vmem_limit_bytes=48 * 1024 * 1024)


@jax.jit
def _forward(state, slab):
    B = state.shape[0]
    b_pad = ((B + 7) // 8) * 8
    tb = next(t for t in (4096, 2048, 1024, 512, 256, 128) if b_pad % t == 0) \
        if b_pad % 128 == 0 else b_pad
    nb = b_pad // tb
    tb2 = next(t for t in (4096, 2048, 1024, 512, 256, 128) if b_pad % t == 0) \
        if b_pad % 128 == 0 else b_pad
    nb2 = b_pad // tb2
    if b_pad != B:
        state = jnp.pad(state, ((0, b_pad - B), (0, 0)))
    n_valid = None if b_pad == B else B
    lane = jnp.arange(LANES)[None, :]

    w1u = jnp.take(slab[0:48, 0:32], _W1_IDX, axis=0)
    w1t = w1u.reshape(25, 128, 32).transpose(0, 2, 1).reshape(800, 128)
    w1t = jnp.where(lane == 125, jnp.tile(slab[B1_ROW, 0:32], 25)[:, None], w1t)
    w2t = slab[W2_OFF:WFC_OFF, 0:32].reshape(3, 96, 32)
    w2t = w2t.transpose(0, 2, 1).reshape(96, 96)
    w2t = jnp.pad(w2t, ((0, 0), (0, 32)))
    w2t = jnp.where(lane == 96, jnp.tile(slab[B2_ROW, 0:32], 3)[:, None], w2t)
    wfct = jnp.transpose(slab[WFC_OFF:WFC_OFF + 32, :])
    wfct = jnp.where(jnp.arange(32)[None, :] == 30, slab[BFC_ROW, :][:, None],
                     wfct)
    wfct = jnp.pad(wfct, ((0, 0), (0, 96)))
    wts = jnp.concatenate([w1t, w2t, wfct], axis=0)

    flops = 2 * b_pad * (800 * LANES + 12 * 32 * 96 + LANES * 32)
    bytes1 = 4 * (b_pad * 125 + 1024 * LANES + b_pad * LANES + nb * LANES * LANES)
    y, stats = pl.pallas_call(
        functools.partial(_pass1, tb=tb, n_valid=n_valid),
        grid=(nb,),
        in_specs=[pl.BlockSpec((tb, 125), lambda i: (i, 0)),
                  pl.BlockSpec((1024, LANES), lambda i: (0, 0))],
        out_specs=[pl.BlockSpec((LANES, tb), lambda i: (0, i)),
                   pl.BlockSpec((LANES, LANES), lambda i: (0, i))],
        out_shape=(jax.ShapeDtypeStruct((LANES, b_pad), jnp.bfloat16),
                   jax.ShapeDtypeStruct((LANES, nb * LANES), jnp.float32)),
        scratch_shapes=[pltpu.VMEM((512, tb), jnp.float32)],
        compiler_params=pltpu.CompilerParams(
            dimension_semantics=("parallel",), **_MOSAIC),
        cost_estimate=pl.CostEstimate(flops=flops, transcendentals=0,
                                      bytes_accessed=bytes1),
    )(state, wts)

    totals = jnp.sum(stats.reshape(LANES, nb, LANES), axis=1)
    aux = jnp.concatenate(
        [totals[:, 0:1], totals[:, 1:2],
         slab[GAMMA_ROW, :][:, None], slab[BETA_ROW, :][:, None]], axis=1)
    aux = jnp.pad(aux, ((0, 0), (0, LANES - 4)))

    out = pl.pallas_call(
        functools.partial(_pass2, inv_n=1.0 / B),
        grid=(nb2,),
        in_specs=[pl.BlockSpec((LANES, tb2), lambda i: (0, i)),
                  pl.BlockSpec((LANES, LANES), lambda i: (0, 0))],
        out_specs=pl.BlockSpec((tb2, FEAT), lambda i: (i, 0)),
        out_shape=jax.ShapeDtypeStruct((b_pad, FEAT), jnp.float32),
        compiler_params=pltpu.CompilerParams(
            dimension_semantics=("parallel",), **_MOSAIC),
        cost_estimate=pl.CostEstimate(
            flops=10 * b_pad * LANES, transcendentals=LANES,
            bytes_accessed=4 * (b_pad * LANES + b_pad * FEAT)),
    )(y, aux)

    return out[:B] if b_pad != B else out


def kernel(state, slab):
    return _forward(state, slab)
